# baseline (speedup 1.0000x reference)
import functools

import jax
import jax.numpy as jnp
from jax import lax
from jax.experimental import pallas as pl
from jax.experimental.pallas import tpu as pltpu
from jax.experimental.pallas import tpu_sc as plsc

WIN = 32
AREA = WIN * WIN
B = AREA * AREA
H = 16
TBL = (2 * WIN - 1) ** 2
TBLP = TBL + 7

NC, NS, L = 2, 16, 16
NW = NC * NS
HG = 8
GROUPS = H // HG
NSLICE = NW // GROUPS
PER_W = B // NSLICE
C = 1024
CR = C // AREA
NCH = PER_W // C
NBUF = 8
VPR = AREA // L


def _sc_body(tbl_hbm, idx_hbm, out_hbm, tbl_v, idx_v, out_v,
             tbl_sem, *sems):
    isems = sems[:NBUF]
    osems = sems[NBUF:]
    wid = lax.axis_index("s") * NC + lax.axis_index("c")
    g = wid % GROUPS
    ps = wid // GROUPS
    row_base = ps * (PER_W // AREA)

    tbl_cp = pltpu.async_copy(tbl_hbm.at[pl.ds(g * HG * TBLP, HG * TBLP)],
                              tbl_v, tbl_sem)

    def fire_idx(s, b):
        pltpu.async_copy(idx_hbm.at[pl.ds(row_base + s * CR, CR), :],
                         idx_v.at[b], isems[b])

    def wait_idx(b):
        pltpu.make_async_copy(idx_hbm.at[pl.ds(0, CR), :], idx_v.at[b],
                              isems[b]).wait()

    def fire_out(s, b):
        pltpu.async_copy(
            out_v.at[b],
            out_hbm.at[pl.ds(g * HG, HG), pl.ds(row_base + s * CR, CR), :],
            osems[b])

    def wait_out(b):
        pltpu.make_async_copy(out_hbm.at[pl.ds(0, HG), pl.ds(0, CR), :],
                              out_v.at[b], osems[b]).wait()

    def compute(b):
        for r in range(CR):
            @plsc.parallel_loop(0, VPR, step=1, unroll=4)
            def vbody(v):
                iv = idx_v[b, r, pl.ds(v * L, L)]
                for h in range(HG):
                    out_v[b, h, r, pl.ds(v * L, L)] = plsc.load_gather(
                        tbl_v, [iv + jnp.int32(h * TBLP)])

    for b in range(NBUF):
        fire_idx(b, b)
    tbl_cp.wait()

    for b in range(NBUF):
        wait_idx(b)
        compute(b)
        fire_out(b, b)
        fire_idx(b + NBUF, b)

    @pl.loop(NBUF, NCH, step=NBUF)
    def _chunks(s):
        for b in range(NBUF):
            sb = s + b
            wait_idx(b)
            wait_out(b)
            compute(b)
            fire_out(sb, b)

            @pl.when(sb + NBUF < NCH)
            def _():
                fire_idx(sb + NBUF, b)

    for b in range(NBUF):
        wait_out(b)


@functools.cache
def _build():
    mesh = plsc.VectorSubcoreMesh(core_axis_name="c", subcore_axis_name="s")
    return pl.kernel(
        _sc_body,
        out_type=jax.ShapeDtypeStruct((H, AREA, AREA), jnp.float32),
        mesh=mesh,
        compiler_params=pltpu.CompilerParams(needs_layout_passes=False),
        scratch_types=[
            pltpu.VMEM((HG * TBLP,), jnp.float32),
            pltpu.VMEM((NBUF, CR, AREA), jnp.int32),
            pltpu.VMEM((NBUF, HG, CR, AREA), jnp.float32),
        ] + [pltpu.SemaphoreType.DMA] * (1 + 2 * NBUF),
    )


def kernel(attn_area, relative_position_bias_table, relative_position_index):
    del attn_area
    tbl = jnp.pad(relative_position_bias_table.T, ((0, 0), (0, TBLP - TBL)))
    out = _build()(tbl.reshape(H * TBLP), relative_position_index)
    return out[None]


# speedup vs baseline: 1.0181x; 1.0181x over previous
import functools

import jax
import jax.numpy as jnp
from jax import lax
from jax.experimental import pallas as pl
from jax.experimental.pallas import tpu as pltpu
from jax.experimental.pallas import tpu_sc as plsc

WIN = 32
AREA = WIN * WIN
B = AREA * AREA
H = 16
TBL = (2 * WIN - 1) ** 2
TBLP = TBL + 7

NC, NS, L = 2, 16, 16
NW = NC * NS
HG = 8
GROUPS = H // HG
NSLICE = NW // GROUPS
PER_W = B // NSLICE
C = 2048
CR = C // AREA
NCH = PER_W // C
NBUF = 4
VPR = AREA // L


def _sc_body(tbl_hbm, idx_hbm, out_hbm, tbl_v, idx_v, out_v,
             tbl_sem, *sems):
    isems = sems[:NBUF]
    osems = sems[NBUF:]
    wid = lax.axis_index("s") * NC + lax.axis_index("c")
    g = wid % GROUPS
    ps = wid // GROUPS
    row_base = ps * (PER_W // AREA)

    tbl_cp = pltpu.async_copy(tbl_hbm.at[pl.ds(g * HG * TBLP, HG * TBLP)],
                              tbl_v, tbl_sem)

    def fire_idx(s, b):
        pltpu.async_copy(idx_hbm.at[pl.ds(row_base + s * CR, CR), :],
                         idx_v.at[b], isems[b])

    def wait_idx(b):
        pltpu.make_async_copy(idx_hbm.at[pl.ds(0, CR), :], idx_v.at[b],
                              isems[b]).wait()

    def fire_out(s, b):
        pltpu.async_copy(
            out_v.at[b],
            out_hbm.at[pl.ds(g * HG, HG), pl.ds(row_base + s * CR, CR), :],
            osems[b])

    def wait_out(b):
        pltpu.make_async_copy(out_hbm.at[pl.ds(0, HG), pl.ds(0, CR), :],
                              out_v.at[b], osems[b]).wait()

    def compute(b):
        for r in range(CR):
            @plsc.parallel_loop(0, VPR, step=1, unroll=4)
            def vbody(v):
                iv = idx_v[b, r, pl.ds(v * L, L)]
                for h in range(HG):
                    out_v[b, h, r, pl.ds(v * L, L)] = plsc.load_gather(
                        tbl_v, [iv + jnp.int32(h * TBLP)])

    for b in range(NBUF):
        fire_idx(b, b)
    tbl_cp.wait()

    for b in range(NBUF):
        wait_idx(b)
        compute(b)
        fire_out(b, b)
        fire_idx(b + NBUF, b)

    @pl.loop(NBUF, NCH, step=NBUF)
    def _chunks(s):
        for b in range(NBUF):
            sb = s + b
            wait_idx(b)
            wait_out(b)
            compute(b)
            fire_out(sb, b)

            @pl.when(sb + NBUF < NCH)
            def _():
                fire_idx(sb + NBUF, b)

    for b in range(NBUF):
        wait_out(b)


@functools.cache
def _build():
    mesh = plsc.VectorSubcoreMesh(core_axis_name="c", subcore_axis_name="s")
    return pl.kernel(
        _sc_body,
        out_type=jax.ShapeDtypeStruct((H, AREA, AREA), jnp.float32),
        mesh=mesh,
        compiler_params=pltpu.CompilerParams(needs_layout_passes=False),
        scratch_types=[
            pltpu.VMEM((HG * TBLP,), jnp.float32),
            pltpu.VMEM((NBUF, CR, AREA), jnp.int32),
            pltpu.VMEM((NBUF, HG, CR, AREA), jnp.float32),
        ] + [pltpu.SemaphoreType.DMA] * (1 + 2 * NBUF),
    )


def kernel(attn_area, relative_position_bias_table, relative_position_index):
    del attn_area
    tbl = jnp.pad(relative_position_bias_table.T, ((0, 0), (0, TBLP - TBL)))
    out = _build()(tbl.reshape(H * TBLP), relative_position_index)
    return out[None]
